# baseline (speedup 1.0000x reference)
import math

import functools

import jax
import jax.numpy as jnp
from jax.experimental import pallas as pl
from jax.experimental.pallas import tpu as pltpu

_B = 8
_IMG = 224
_PATCH = 16
_GRID = _IMG // _PATCH
_NPATCH = _GRID * _GRID
_S = _NPATCH + 1
_D = 768
_H = 12
_HD = _D // _H
_DEPTH = 12
_PRUNE_AFTER = 4
_MLP = 4 * _D
_NCLS = 1000
_NKEEP = 98
_KP = 104
_PDIM = 3 * _PATCH * _PATCH
_EPS = 1e-6
_ISQRT2 = 0.7071067811865476
_ASCALE = 1.0 / math.sqrt(_HD)


def _mm(a, b):
    return jax.lax.dot_general(a, b, (((1,), (0,)), ((), ())),
                               preferred_element_type=jnp.float32)


def _mmT(a, b):
    return jax.lax.dot_general(a, b, (((1,), (1,)), ((), ())),
                               preferred_element_type=jnp.float32)


def _ln(x, g, b):
    m = jnp.mean(x, axis=-1, keepdims=True)
    xc = x - m
    v = jnp.mean(xc * xc, axis=-1, keepdims=True)
    return xc * jax.lax.rsqrt(v + _EPS) * g + b


def _gelu(x):
    return 0.5 * x * (1.0 + jax.lax.erf(x * _ISQRT2))


def _front_kernel(p_ref, pw_ref, pb_ref, cls_ref, pos0_ref, posr_ref, out_ref):
    emb = _mmT(p_ref[0], pw_ref[...]) + pb_ref[...] + posr_ref[...]
    row0 = cls_ref[...] + pos0_ref[...]
    out_ref[0] = jnp.concatenate([row0, emb], axis=0)


def _front(p, patch_w, patch_b, cls_tok, pos0, posr):
    return pl.pallas_call(
        _front_kernel,
        grid=(_B,),
        in_specs=[
            pl.BlockSpec((1, _NPATCH, _PDIM), lambda b: (b, 0, 0)),
            pl.BlockSpec((_D, _PDIM), lambda b: (0, 0)),
            pl.BlockSpec((1, _D), lambda b: (0, 0)),
            pl.BlockSpec((1, _D), lambda b: (0, 0)),
            pl.BlockSpec((1, _D), lambda b: (0, 0)),
            pl.BlockSpec((_NPATCH, _D), lambda b: (0, 0)),
        ],
        out_specs=pl.BlockSpec((1, _S, _D), lambda b: (b, 0, 0)),
        out_shape=jax.ShapeDtypeStruct((_B, _S, _D), jnp.float32),
    )(p, patch_w, patch_b, cls_tok, pos0, posr)


def _block_kernel(x_any, m_ref, g1_ref, b1_ref, qw_ref, qb_ref, pw_ref, pb_ref,
                  g2_ref, b2_ref, w1_ref, c1_ref, w2_ref, c2_ref, out_any,
                  xbuf, in_sem, out_sem, *, masked):
    b = pl.program_id(1)
    del x_any
    cp_in = pltpu.make_async_copy(out_any.at[b], xbuf, in_sem)
    cp_in.start()
    cp_in.wait()
    x = xbuf[...]
    h = _ln(x, g1_ref[0], b1_ref[0])
    qkv = _mmT(h, qw_ref[0]) + qb_ref[0]
    neg = (1.0 - m_ref[0]) * (-1e30) if masked else None
    outs = []
    for i in range(_H):
        qh = qkv[:, i * _HD:(i + 1) * _HD]
        kh = qkv[:, _D + i * _HD:_D + (i + 1) * _HD]
        vh = qkv[:, 2 * _D + i * _HD:2 * _D + (i + 1) * _HD]
        l = _mmT(qh, kh) * _ASCALE
        if masked:
            l = l + neg
        l = l - jnp.max(l, axis=-1, keepdims=True)
        e = jnp.exp(l)
        a = e / jnp.sum(e, axis=-1, keepdims=True)
        outs.append(_mm(a, vh))
    o = jnp.concatenate(outs, axis=1)
    x = x + _mmT(o, pw_ref[0]) + pb_ref[0]
    h2 = _ln(x, g2_ref[0], b2_ref[0])
    mh = _gelu(_mmT(h2, w1_ref[0]) + c1_ref[0])
    xbuf[...] = x + _mmT(mh, w2_ref[0]) + c2_ref[0]
    cp_out = pltpu.make_async_copy(xbuf, out_any.at[b], out_sem)
    cp_out.start()
    cp_out.wait()


def _run_phase(x, mask, l0, nlayers, masked, n1g, n1b, qkv_w, qkv_b, proj_w,
               proj_b, n2g, n2b, fc1_w, fc1_b, fc2_w, fc2_b):
    seq = x.shape[1]

    def w3(shape):
        return pl.BlockSpec((1,) + shape, lambda l, b: (l0 + l, 0, 0))

    return pl.pallas_call(
        functools.partial(_block_kernel, masked=masked),
        grid=(nlayers, _B),
        in_specs=[
            pl.BlockSpec(memory_space=pl.ANY),
            pl.BlockSpec((1, 1, seq), lambda l, b: (b, 0, 0)),
            w3((1, _D)), w3((1, _D)),
            w3((3 * _D, _D)), w3((1, 3 * _D)),
            w3((_D, _D)), w3((1, _D)),
            w3((1, _D)), w3((1, _D)),
            w3((_MLP, _D)), w3((1, _MLP)),
            w3((_D, _MLP)), w3((1, _D)),
        ],
        out_specs=pl.BlockSpec(memory_space=pl.ANY),
        out_shape=jax.ShapeDtypeStruct((_B, seq, _D), jnp.float32),
        input_output_aliases={0: 0},
        scratch_shapes=[
            pltpu.VMEM((seq, _D), jnp.float32),
            pltpu.SemaphoreType.DMA,
            pltpu.SemaphoreType.DMA,
        ],
    )(x, mask, n1g, n1b, qkv_w, qkv_b, proj_w, proj_b,
      n2g, n2b, fc1_w, fc1_b, fc2_w, fc2_b)


def _transpose_col(col, n):
    i_col = jax.lax.broadcasted_iota(jnp.int32, (n, 1), 0)
    j_row = jax.lax.broadcasted_iota(jnp.int32, (1, n), 1)
    return jnp.sum(jnp.where(i_col == j_row, col, 0.0), axis=0, keepdims=True)


def _pack_kernel(x_ref, out_ref, valid_ref):
    x = x_ref[0]
    xx = x * x
    s_col = jnp.sqrt(_mm(xx, jnp.ones((_D, 1), jnp.float32)))
    s_row = _transpose_col(s_col, _S)
    i_col = jax.lax.broadcasted_iota(jnp.int32, (_S, 1), 0)
    j_row = jax.lax.broadcasted_iota(jnp.int32, (1, _S), 1)
    beats = (s_row > s_col) | ((s_row == s_col) & (j_row < i_col))
    rank = jnp.sum(beats.astype(jnp.float32), axis=1, keepdims=True)
    keep_col = ((rank < float(_NKEEP)) | (i_col == 0)).astype(jnp.float32)
    keep_row = _transpose_col(keep_col, _S)
    count = jnp.sum(keep_col)
    below = (j_row < i_col).astype(jnp.float32)
    pos_col = jnp.sum(keep_row * below, axis=1, keepdims=True)
    pos_row = _transpose_col(pos_col, _S)
    p_col = jax.lax.broadcasted_iota(jnp.int32, (_KP, 1), 0).astype(jnp.float32)
    sel = ((p_col == pos_row) & (keep_row > 0.5)).astype(jnp.float32)
    out_ref[0] = _mm(sel, x)
    kp_row = jax.lax.broadcasted_iota(jnp.int32, (1, _KP), 1).astype(jnp.float32)
    valid_ref[0] = (kp_row < count).astype(jnp.float32)


def _pack(x):
    return pl.pallas_call(
        _pack_kernel,
        grid=(_B,),
        in_specs=[pl.BlockSpec((1, _S, _D), lambda b: (b, 0, 0))],
        out_specs=[
            pl.BlockSpec((1, _KP, _D), lambda b: (b, 0, 0)),
            pl.BlockSpec((1, 1, _KP), lambda b: (b, 0, 0)),
        ],
        out_shape=[
            jax.ShapeDtypeStruct((_B, _KP, _D), jnp.float32),
            jax.ShapeDtypeStruct((_B, 1, _KP), jnp.float32),
        ],
    )(x)


def _head_kernel(x_ref, g_ref, b_ref, w_ref, hb_ref, out_ref):
    h = _ln(x_ref[...], g_ref[...], b_ref[...])
    out_ref[...] = _mmT(h, w_ref[...]) + hb_ref[...]


def _head(cls, norm_g, norm_b, head_w, head_b):
    return pl.pallas_call(
        _head_kernel,
        out_shape=jax.ShapeDtypeStruct((_B, _NCLS), jnp.float32),
    )(cls, norm_g, norm_b, head_w, head_b)


def kernel(images, patch_w, patch_b, cls_token, pos_embed, n1g, n1b, qkv_w,
           qkv_b, proj_w, proj_b, n2g, n2b, fc1_w, fc1_b, fc2_w, fc2_b,
           norm_g, norm_b, head_w, head_b):
    p = images.reshape(_B, 3, _GRID, _PATCH, _GRID, _PATCH)
    p = p.transpose(0, 2, 4, 1, 3, 5).reshape(_B, _NPATCH, _PDIM)
    pos = pos_embed.reshape(_S, _D)
    x = _front(p, patch_w, patch_b.reshape(1, _D), cls_token.reshape(1, _D),
               pos[0:1], pos[1:])

    n1g3 = n1g.reshape(_DEPTH, 1, _D)
    n1b3 = n1b.reshape(_DEPTH, 1, _D)
    qkv_b3 = qkv_b.reshape(_DEPTH, 1, 3 * _D)
    proj_b3 = proj_b.reshape(_DEPTH, 1, _D)
    n2g3 = n2g.reshape(_DEPTH, 1, _D)
    n2b3 = n2b.reshape(_DEPTH, 1, _D)
    fc1_b3 = fc1_b.reshape(_DEPTH, 1, _MLP)
    fc2_b3 = fc2_b.reshape(_DEPTH, 1, _D)

    def phase(xx, mask, l0, nlayers, masked):
        return _run_phase(xx, mask, l0, nlayers, masked, n1g3, n1b3, qkv_w,
                          qkv_b3, proj_w, proj_b3, n2g3, n2b3, fc1_w, fc1_b3,
                          fc2_w, fc2_b3)

    dense_mask = jnp.ones((_B, 1, _S), jnp.float32)
    x = phase(x, dense_mask, 0, _PRUNE_AFTER, masked=False)

    packed, valid = _pack(x)
    packed = phase(packed, valid, _PRUNE_AFTER, _DEPTH - _PRUNE_AFTER,
                   masked=True)

    cls = packed[:, 0, :]
    return _head(cls, norm_g.reshape(1, _D), norm_b.reshape(1, _D),
                 head_w, head_b.reshape(1, _NCLS))


# speedup vs baseline: 1.1081x; 1.1081x over previous
import math

import functools

import jax
import jax.numpy as jnp
from jax.experimental import pallas as pl
from jax.experimental.pallas import tpu as pltpu

_B = 8
_IMG = 224
_PATCH = 16
_GRID = _IMG // _PATCH
_NPATCH = _GRID * _GRID
_S = _NPATCH + 1
_D = 768
_H = 12
_HD = _D // _H
_DEPTH = 12
_PRUNE_AFTER = 4
_MLP = 4 * _D
_NCLS = 1000
_NKEEP = 98
_KP = 104
_PDIM = 3 * _PATCH * _PATCH
_EPS = 1e-6
_ISQRT2 = 0.7071067811865476
_ASCALE = 1.0 / math.sqrt(_HD)


def _mm(a, b, precision=None):
    return jax.lax.dot_general(a, b, (((1,), (0,)), ((), ())),
                               preferred_element_type=jnp.float32,
                               precision=precision)


def _mmT(a, b, precision=None):
    return jax.lax.dot_general(a, b, (((1,), (1,)), ((), ())),
                               preferred_element_type=jnp.float32,
                               precision=precision)


def _split(a):
    hi = a.astype(jnp.bfloat16)
    lo = (a - hi.astype(jnp.float32)).astype(jnp.bfloat16)
    return hi, lo


def _mm3T(a, w):
    return _mmT(a.astype(jnp.bfloat16), w.astype(jnp.bfloat16))


def _mm3(a, w):
    return _mm(a.astype(jnp.bfloat16), w.astype(jnp.bfloat16))


def _ln(x, g, b):
    m = jnp.mean(x, axis=-1, keepdims=True)
    xc = x - m
    v = jnp.mean(xc * xc, axis=-1, keepdims=True)
    return xc / jnp.sqrt(v + _EPS) * g + b


def _gelu(x):
    return 0.5 * x * (1.0 + jax.lax.erf(x * _ISQRT2))


def _front_kernel(p_ref, pw_ref, pb_ref, cls_ref, pos0_ref, posr_ref, out_ref):
    emb = _mm3T(p_ref[0], pw_ref[...]) + pb_ref[...] + posr_ref[...]
    row0 = cls_ref[...] + pos0_ref[...]
    out_ref[0] = jnp.concatenate([row0, emb], axis=0)


def _front(p, patch_w, patch_b, cls_tok, pos0, posr):
    return pl.pallas_call(
        _front_kernel,
        grid=(_B,),
        in_specs=[
            pl.BlockSpec((1, _NPATCH, _PDIM), lambda b: (b, 0, 0)),
            pl.BlockSpec((_D, _PDIM), lambda b: (0, 0)),
            pl.BlockSpec((1, _D), lambda b: (0, 0)),
            pl.BlockSpec((1, _D), lambda b: (0, 0)),
            pl.BlockSpec((1, _D), lambda b: (0, 0)),
            pl.BlockSpec((_NPATCH, _D), lambda b: (0, 0)),
        ],
        out_specs=pl.BlockSpec((1, _S, _D), lambda b: (b, 0, 0)),
        out_shape=jax.ShapeDtypeStruct((_B, _S, _D), jnp.float32),
    )(p, patch_w, patch_b, cls_tok, pos0, posr)


def _phase_kernel(x_any, m_ref, g1_ref, b1_ref, qb_ref, pb_ref, g2_ref,
                  b2_ref, c1_ref, c2_ref, qw_any, pw_any, w1_any, w2_any,
                  out_any,
                  xbuf, qwbuf, pwbuf, w1buf, w2buf,
                  in_sem, out_sem, qw_sem, pw_sem, w1_sem, w2_sem,
                  *, masked, l0, nlayers):
    del x_any
    l = pl.program_id(0)
    b = pl.program_id(1)
    s = l * _B + b
    last_s = nlayers * _B - 1
    slot = jax.lax.rem(l, 2)
    xs = jax.lax.rem(s, 2)

    def wcopies(layer_idx, wslot):
        return [
            pltpu.make_async_copy(qw_any.at[layer_idx], qwbuf.at[wslot],
                                  qw_sem.at[wslot]),
            pltpu.make_async_copy(w1_any.at[layer_idx], w1buf.at[wslot],
                                  w1_sem.at[wslot]),
            pltpu.make_async_copy(w2_any.at[layer_idx], w2buf.at[wslot],
                                  w2_sem.at[wslot]),
        ]

    def pcopy(layer_idx):
        return pltpu.make_async_copy(pw_any.at[layer_idx], pwbuf, pw_sem)

    def xcopy(bb, xslot):
        return pltpu.make_async_copy(out_any.at[bb], xbuf.at[xslot],
                                     in_sem.at[xslot])

    def ocopy(bb, oslot):
        return pltpu.make_async_copy(xbuf.at[oslot], out_any.at[bb],
                                     out_sem.at[oslot])

    @pl.when(s == 0)
    def _():
        for c in wcopies(l0, 0):
            c.start()
        pcopy(l0).start()
        xcopy(0, 0).start()

    @pl.when((b == 0) & (l > 0))
    def _():
        pcopy(l0 + l).start()

    @pl.when(b == 0)
    def _():
        for c in wcopies(l0 + l, slot):
            c.wait()
        pcopy(l0 + l).wait()

    @pl.when((b == 0) & (l + 1 < nlayers))
    def _():
        for c in wcopies(l0 + l + 1, 1 - slot):
            c.start()

    xcopy(b, xs).wait()

    @pl.when(s < last_s)
    def _():

        @pl.when(s >= 1)
        def _():
            ocopy(b, 1 - xs).wait()

        nb = jnp.where(b + 1 < _B, b + 1, 0)
        xcopy(nb, 1 - xs).start()

    x = xbuf[xs]
    h = _ln(x, g1_ref[0], b1_ref[0])
    qkv = _mm3T(h, qwbuf[slot]) + qb_ref[0]
    neg = (1.0 - m_ref[0]) * (-1e30) if masked else None
    outs = []
    for i in range(_H):
        qh = qkv[:, i * _HD:(i + 1) * _HD]
        kh = qkv[:, _D + i * _HD:_D + (i + 1) * _HD]
        vh = qkv[:, 2 * _D + i * _HD:2 * _D + (i + 1) * _HD]
        lg = _mm3T(qh, kh) * _ASCALE
        if masked:
            lg = lg + neg
        lg = lg - jnp.max(lg, axis=-1, keepdims=True)
        e = jnp.exp(lg)
        a = e / jnp.sum(e, axis=-1, keepdims=True)
        outs.append(_mm3(a, vh))
    o = jnp.concatenate(outs, axis=1)
    x = x + _mm3T(o, pwbuf[...]) + pb_ref[0]
    h2 = _ln(x, g2_ref[0], b2_ref[0])
    mh = _gelu(_mm3T(h2, w1buf[slot]) + c1_ref[0])
    xbuf[xs] = x + _mm3T(mh, w2buf[slot]) + c2_ref[0]
    ocopy(b, xs).start()

    @pl.when(s == last_s)
    def _():
        ocopy(b, xs).wait()
        ocopy(b - 1, 1 - xs).wait()


def _run_phase(x, mask, l0, nlayers, masked, n1g, n1b, qkv_w, qkv_b, proj_w,
               proj_b, n2g, n2b, fc1_w, fc1_b, fc2_w, fc2_b):
    seq = x.shape[1]

    def w3(shape):
        return pl.BlockSpec((1,) + shape, lambda l, b: (l0 + l, 0, 0))

    anyspec = pl.BlockSpec(memory_space=pl.ANY)
    return pl.pallas_call(
        functools.partial(_phase_kernel, masked=masked, l0=l0,
                          nlayers=nlayers),
        grid=(nlayers, _B),
        in_specs=[
            anyspec,
            pl.BlockSpec((1, 1, seq), lambda l, b: (b, 0, 0)),
            w3((1, _D)), w3((1, _D)), w3((1, 3 * _D)), w3((1, _D)),
            w3((1, _D)), w3((1, _D)), w3((1, _MLP)), w3((1, _D)),
            anyspec, anyspec, anyspec, anyspec,
        ],
        out_specs=anyspec,
        out_shape=jax.ShapeDtypeStruct((_B, seq, _D), jnp.float32),
        input_output_aliases={0: 0},
        scratch_shapes=[
            pltpu.VMEM((2, seq, _D), jnp.float32),
            pltpu.VMEM((2, 3 * _D, _D), jnp.float32),
            pltpu.VMEM((_D, _D), jnp.float32),
            pltpu.VMEM((2, _MLP, _D), jnp.float32),
            pltpu.VMEM((2, _D, _MLP), jnp.float32),
            pltpu.SemaphoreType.DMA((2,)),
            pltpu.SemaphoreType.DMA((2,)),
            pltpu.SemaphoreType.DMA((2,)),
            pltpu.SemaphoreType.DMA,
            pltpu.SemaphoreType.DMA((2,)),
            pltpu.SemaphoreType.DMA((2,)),
        ],
        compiler_params=pltpu.CompilerParams(
            vmem_limit_bytes=100 * 1024 * 1024,
        ),
    )(x, mask, n1g, n1b, qkv_b, proj_b, n2g, n2b, fc1_b, fc2_b,
      qkv_w, proj_w, fc1_w, fc2_w)


def _transpose_col(col, n):
    i_col = jax.lax.broadcasted_iota(jnp.int32, (n, 1), 0)
    j_row = jax.lax.broadcasted_iota(jnp.int32, (1, n), 1)
    return jnp.sum(jnp.where(i_col == j_row, col, 0.0), axis=0, keepdims=True)


def _pack_kernel(x_ref, out_ref, valid_ref):
    x = x_ref[0]
    s_col = jnp.sqrt(jnp.sum(x * x, axis=1, keepdims=True))
    s_row = _transpose_col(s_col, _S)
    i_col = jax.lax.broadcasted_iota(jnp.int32, (_S, 1), 0)
    j_row = jax.lax.broadcasted_iota(jnp.int32, (1, _S), 1)
    beats = (s_row > s_col) | ((s_row == s_col) & (j_row < i_col))
    rank = jnp.sum(beats.astype(jnp.float32), axis=1, keepdims=True)
    keep_col = ((rank < float(_NKEEP)) | (i_col == 0)).astype(jnp.float32)
    keep_row = _transpose_col(keep_col, _S)
    count = jnp.sum(keep_col)
    below = (j_row < i_col).astype(jnp.float32)
    pos_col = jnp.sum(keep_row * below, axis=1, keepdims=True)
    pos_row = _transpose_col(pos_col, _S)
    p_col = jax.lax.broadcasted_iota(jnp.int32, (_KP, 1), 0).astype(jnp.float32)
    sel = ((p_col == pos_row) & (keep_row > 0.5)).astype(jnp.float32)
    x_hi, x_lo = _split(x)
    out_ref[0] = _mm(sel, x_hi) + _mm(sel, x_lo)
    kp_row = jax.lax.broadcasted_iota(jnp.int32, (1, _KP), 1).astype(jnp.float32)
    valid_ref[0] = (kp_row < count).astype(jnp.float32)


def _pack(x):
    return pl.pallas_call(
        _pack_kernel,
        grid=(_B,),
        in_specs=[pl.BlockSpec((1, _S, _D), lambda b: (b, 0, 0))],
        out_specs=[
            pl.BlockSpec((1, _KP, _D), lambda b: (b, 0, 0)),
            pl.BlockSpec((1, 1, _KP), lambda b: (b, 0, 0)),
        ],
        out_shape=[
            jax.ShapeDtypeStruct((_B, _KP, _D), jnp.float32),
            jax.ShapeDtypeStruct((_B, 1, _KP), jnp.float32),
        ],
    )(x)


def _head_kernel(x_ref, g_ref, b_ref, w_ref, hb_ref, out_ref):
    h = _ln(x_ref[...], g_ref[...], b_ref[...])
    out_ref[...] = _mm3T(h, w_ref[...]) + hb_ref[...]


def _head(cls, norm_g, norm_b, head_w, head_b):
    return pl.pallas_call(
        _head_kernel,
        out_shape=jax.ShapeDtypeStruct((_B, _NCLS), jnp.float32),
    )(cls, norm_g, norm_b, head_w, head_b)


def kernel(images, patch_w, patch_b, cls_token, pos_embed, n1g, n1b, qkv_w,
           qkv_b, proj_w, proj_b, n2g, n2b, fc1_w, fc1_b, fc2_w, fc2_b,
           norm_g, norm_b, head_w, head_b):
    p = images.reshape(_B, 3, _GRID, _PATCH, _GRID, _PATCH)
    p = p.transpose(0, 2, 4, 1, 3, 5).reshape(_B, _NPATCH, _PDIM)
    pos = pos_embed.reshape(_S, _D)
    x = _front(p, patch_w, patch_b.reshape(1, _D), cls_token.reshape(1, _D),
               pos[0:1], pos[1:])

    n1g3 = n1g.reshape(_DEPTH, 1, _D)
    n1b3 = n1b.reshape(_DEPTH, 1, _D)
    qkv_b3 = qkv_b.reshape(_DEPTH, 1, 3 * _D)
    proj_b3 = proj_b.reshape(_DEPTH, 1, _D)
    n2g3 = n2g.reshape(_DEPTH, 1, _D)
    n2b3 = n2b.reshape(_DEPTH, 1, _D)
    fc1_b3 = fc1_b.reshape(_DEPTH, 1, _MLP)
    fc2_b3 = fc2_b.reshape(_DEPTH, 1, _D)

    def phase(xx, mask, l0, nlayers, masked):
        return _run_phase(xx, mask, l0, nlayers, masked, n1g3, n1b3, qkv_w,
                          qkv_b3, proj_w, proj_b3, n2g3, n2b3, fc1_w, fc1_b3,
                          fc2_w, fc2_b3)

    dense_mask = jnp.ones((_B, 1, _S), jnp.float32)
    x = phase(x, dense_mask, 0, _PRUNE_AFTER, masked=False)

    packed, valid = _pack(x)
    packed = phase(packed, valid, _PRUNE_AFTER, _DEPTH - _PRUNE_AFTER,
                   masked=True)

    cls = packed[:, 0, :]
    return _head(cls, norm_g.reshape(1, _D), norm_b.reshape(1, _D),
                 head_w, head_b.reshape(1, _NCLS))


# speedup vs baseline: 1.1173x; 1.0083x over previous
import math

import functools

import jax
import jax.numpy as jnp
from jax.experimental import pallas as pl
from jax.experimental.pallas import tpu as pltpu

_B = 8
_IMG = 224
_PATCH = 16
_GRID = _IMG // _PATCH
_NPATCH = _GRID * _GRID
_S = _NPATCH + 1
_D = 768
_H = 12
_HD = _D // _H
_DEPTH = 12
_PRUNE_AFTER = 4
_MLP = 4 * _D
_NCLS = 1000
_NKEEP = 98
_KP = 104
_PDIM = 3 * _PATCH * _PATCH
_EPS = 1e-6
_ISQRT2 = 0.7071067811865476
_ASCALE = 1.0 / math.sqrt(_HD)


def _mm(a, b, precision=None):
    return jax.lax.dot_general(a, b, (((1,), (0,)), ((), ())),
                               preferred_element_type=jnp.float32,
                               precision=precision)


def _mmT(a, b, precision=None):
    return jax.lax.dot_general(a, b, (((1,), (1,)), ((), ())),
                               preferred_element_type=jnp.float32,
                               precision=precision)


def _split(a):
    hi = a.astype(jnp.bfloat16)
    lo = (a - hi.astype(jnp.float32)).astype(jnp.bfloat16)
    return hi, lo


def _mm3T(a, w):
    return _mmT(a.astype(jnp.bfloat16), w.astype(jnp.bfloat16))


def _mm3(a, w):
    return _mm(a.astype(jnp.bfloat16), w.astype(jnp.bfloat16))


def _ln(x, g, b):
    m = jnp.mean(x, axis=-1, keepdims=True)
    xc = x - m
    v = jnp.mean(xc * xc, axis=-1, keepdims=True)
    return xc / jnp.sqrt(v + _EPS) * g + b


def _gelu(x):
    return 0.5 * x * (1.0 + jax.lax.erf(x * _ISQRT2))


def _front_kernel(p_ref, pw_ref, pb_ref, cls_ref, pos0_ref, posr_ref, out_ref):
    emb = _mm3T(p_ref[0], pw_ref[...]) + pb_ref[...] + posr_ref[...]
    row0 = cls_ref[...] + pos0_ref[...]
    out_ref[0] = jnp.concatenate([row0, emb], axis=0)


def _front(p, patch_w, patch_b, cls_tok, pos0, posr):
    return pl.pallas_call(
        _front_kernel,
        grid=(_B,),
        in_specs=[
            pl.BlockSpec((1, _NPATCH, _PDIM), lambda b: (b, 0, 0)),
            pl.BlockSpec((_D, _PDIM), lambda b: (0, 0)),
            pl.BlockSpec((1, _D), lambda b: (0, 0)),
            pl.BlockSpec((1, _D), lambda b: (0, 0)),
            pl.BlockSpec((1, _D), lambda b: (0, 0)),
            pl.BlockSpec((_NPATCH, _D), lambda b: (0, 0)),
        ],
        out_specs=pl.BlockSpec((1, _S, _D), lambda b: (b, 0, 0)),
        out_shape=jax.ShapeDtypeStruct((_B, _S, _D), jnp.float32),
    )(p, patch_w, patch_b, cls_tok, pos0, posr)


def _phase_kernel(x_any, m_ref, g1_ref, b1_ref, qb_ref, pb_ref, g2_ref,
                  b2_ref, c1_ref, c2_ref, qw_any, pw_any, w1_any, w2_any,
                  out_any,
                  xbuf, qwbuf, pwbuf, w1buf, w2buf,
                  in_sem, out_sem, qw_sem, pw_sem, w1_sem, w2_sem,
                  *, masked, l0, nlayers):
    del x_any
    l = pl.program_id(0)
    b = pl.program_id(1)
    s = l * _B + b
    last_s = nlayers * _B - 1
    slot = jax.lax.rem(l, 2)
    xs = jax.lax.rem(s, 2)

    def wcopies(layer_idx, wslot):
        return [
            pltpu.make_async_copy(qw_any.at[layer_idx], qwbuf.at[wslot],
                                  qw_sem.at[wslot]),
            pltpu.make_async_copy(pw_any.at[layer_idx], pwbuf.at[wslot],
                                  pw_sem.at[wslot]),
            pltpu.make_async_copy(w1_any.at[layer_idx], w1buf.at[wslot],
                                  w1_sem.at[wslot]),
            pltpu.make_async_copy(w2_any.at[layer_idx], w2buf.at[wslot],
                                  w2_sem.at[wslot]),
        ]

    def xcopy(bb, xslot):
        return pltpu.make_async_copy(out_any.at[bb], xbuf.at[xslot],
                                     in_sem.at[xslot])

    def ocopy(bb, oslot):
        return pltpu.make_async_copy(xbuf.at[oslot], out_any.at[bb],
                                     out_sem.at[oslot])

    @pl.when(s == 0)
    def _():
        for c in wcopies(l0, 0):
            c.start()
        xcopy(0, 0).start()

    @pl.when(b == 0)
    def _():
        for c in wcopies(l0 + l, slot):
            c.wait()

    @pl.when((b == 0) & (l + 1 < nlayers))
    def _():
        for c in wcopies(l0 + l + 1, 1 - slot):
            c.start()

    xcopy(b, xs).wait()

    @pl.when(s < last_s)
    def _():

        @pl.when(s >= 1)
        def _():
            ocopy(b, 1 - xs).wait()

        nb = jnp.where(b + 1 < _B, b + 1, 0)
        xcopy(nb, 1 - xs).start()

    x = xbuf[xs]
    h = _ln(x, g1_ref[0], b1_ref[0])
    qkv = _mm3T(h, qwbuf[slot]) + qb_ref[0]
    neg = (1.0 - m_ref[0]) * (-1e30) if masked else None
    outs = []
    for i in range(_H):
        qh = qkv[:, i * _HD:(i + 1) * _HD]
        kh = qkv[:, _D + i * _HD:_D + (i + 1) * _HD]
        vh = qkv[:, 2 * _D + i * _HD:2 * _D + (i + 1) * _HD]
        lg = _mm3T(qh, kh) * _ASCALE
        if masked:
            lg = lg + neg
        lg = lg - jnp.max(lg, axis=-1, keepdims=True)
        e = jnp.exp(lg)
        a = e / jnp.sum(e, axis=-1, keepdims=True)
        outs.append(_mm3(a, vh))
    o = jnp.concatenate(outs, axis=1)
    x = x + _mm3T(o, pwbuf[slot]) + pb_ref[0]
    h2 = _ln(x, g2_ref[0], b2_ref[0])
    mh = _gelu(_mm3T(h2, w1buf[slot]) + c1_ref[0])
    xbuf[xs] = x + _mm3T(mh, w2buf[slot]) + c2_ref[0]
    ocopy(b, xs).start()

    @pl.when(s == last_s)
    def _():
        ocopy(b, xs).wait()
        ocopy(b - 1, 1 - xs).wait()


def _run_phase(x, mask, l0, nlayers, masked, n1g, n1b, qkv_w, qkv_b, proj_w,
               proj_b, n2g, n2b, fc1_w, fc1_b, fc2_w, fc2_b):
    seq = x.shape[1]

    def w3(shape):
        return pl.BlockSpec((1,) + shape, lambda l, b: (l0 + l, 0, 0))

    anyspec = pl.BlockSpec(memory_space=pl.ANY)
    return pl.pallas_call(
        functools.partial(_phase_kernel, masked=masked, l0=l0,
                          nlayers=nlayers),
        grid=(nlayers, _B),
        in_specs=[
            anyspec,
            pl.BlockSpec((1, 1, seq), lambda l, b: (b, 0, 0)),
            w3((1, _D)), w3((1, _D)), w3((1, 3 * _D)), w3((1, _D)),
            w3((1, _D)), w3((1, _D)), w3((1, _MLP)), w3((1, _D)),
            anyspec, anyspec, anyspec, anyspec,
        ],
        out_specs=anyspec,
        out_shape=jax.ShapeDtypeStruct((_B, seq, _D), jnp.float32),
        input_output_aliases={0: 0},
        scratch_shapes=[
            pltpu.VMEM((2, seq, _D), jnp.float32),
            pltpu.VMEM((2, 3 * _D, _D), jnp.float32),
            pltpu.VMEM((2, _D, _D), jnp.float32),
            pltpu.VMEM((2, _MLP, _D), jnp.float32),
            pltpu.VMEM((2, _D, _MLP), jnp.float32),
            pltpu.SemaphoreType.DMA((2,)),
            pltpu.SemaphoreType.DMA((2,)),
            pltpu.SemaphoreType.DMA((2,)),
            pltpu.SemaphoreType.DMA((2,)),
            pltpu.SemaphoreType.DMA((2,)),
            pltpu.SemaphoreType.DMA((2,)),
        ],
        compiler_params=pltpu.CompilerParams(
            vmem_limit_bytes=100 * 1024 * 1024,
        ),
    )(x, mask, n1g, n1b, qkv_b, proj_b, n2g, n2b, fc1_b, fc2_b,
      qkv_w, proj_w, fc1_w, fc2_w)


def _transpose_col(col, n):
    i_col = jax.lax.broadcasted_iota(jnp.int32, (n, 1), 0)
    j_row = jax.lax.broadcasted_iota(jnp.int32, (1, n), 1)
    return jnp.sum(jnp.where(i_col == j_row, col, 0.0), axis=0, keepdims=True)


def _pack_kernel(x_ref, out_ref, valid_ref):
    x = x_ref[0]
    s_col = jnp.sqrt(jnp.sum(x * x, axis=1, keepdims=True))
    s_row = _transpose_col(s_col, _S)
    i_col = jax.lax.broadcasted_iota(jnp.int32, (_S, 1), 0)
    j_row = jax.lax.broadcasted_iota(jnp.int32, (1, _S), 1)
    beats = (s_row > s_col) | ((s_row == s_col) & (j_row < i_col))
    rank = jnp.sum(beats.astype(jnp.float32), axis=1, keepdims=True)
    keep_col = ((rank < float(_NKEEP)) | (i_col == 0)).astype(jnp.float32)
    keep_row = _transpose_col(keep_col, _S)
    count = jnp.sum(keep_col)
    below = (j_row < i_col).astype(jnp.float32)
    pos_col = jnp.sum(keep_row * below, axis=1, keepdims=True)
    pos_row = _transpose_col(pos_col, _S)
    p_col = jax.lax.broadcasted_iota(jnp.int32, (_KP, 1), 0).astype(jnp.float32)
    sel = ((p_col == pos_row) & (keep_row > 0.5)).astype(jnp.float32)
    x_hi, x_lo = _split(x)
    out_ref[0] = _mm(sel, x_hi) + _mm(sel, x_lo)
    kp_row = jax.lax.broadcasted_iota(jnp.int32, (1, _KP), 1).astype(jnp.float32)
    valid_ref[0] = (kp_row < count).astype(jnp.float32)


def _pack(x):
    return pl.pallas_call(
        _pack_kernel,
        grid=(_B,),
        in_specs=[pl.BlockSpec((1, _S, _D), lambda b: (b, 0, 0))],
        out_specs=[
            pl.BlockSpec((1, _KP, _D), lambda b: (b, 0, 0)),
            pl.BlockSpec((1, 1, _KP), lambda b: (b, 0, 0)),
        ],
        out_shape=[
            jax.ShapeDtypeStruct((_B, _KP, _D), jnp.float32),
            jax.ShapeDtypeStruct((_B, 1, _KP), jnp.float32),
        ],
    )(x)


def _head_kernel(x_ref, g_ref, b_ref, w_ref, hb_ref, out_ref):
    h = _ln(x_ref[...], g_ref[...], b_ref[...])
    out_ref[...] = _mm3T(h, w_ref[...]) + hb_ref[...]


def _head(cls, norm_g, norm_b, head_w, head_b):
    return pl.pallas_call(
        _head_kernel,
        out_shape=jax.ShapeDtypeStruct((_B, _NCLS), jnp.float32),
    )(cls, norm_g, norm_b, head_w, head_b)


def kernel(images, patch_w, patch_b, cls_token, pos_embed, n1g, n1b, qkv_w,
           qkv_b, proj_w, proj_b, n2g, n2b, fc1_w, fc1_b, fc2_w, fc2_b,
           norm_g, norm_b, head_w, head_b):
    p = images.reshape(_B, 3, _GRID, _PATCH, _GRID, _PATCH)
    p = p.transpose(0, 2, 4, 1, 3, 5).reshape(_B, _NPATCH, _PDIM)
    pos = pos_embed.reshape(_S, _D)
    x = _front(p, patch_w, patch_b.reshape(1, _D), cls_token.reshape(1, _D),
               pos[0:1], pos[1:])

    n1g3 = n1g.reshape(_DEPTH, 1, _D)
    n1b3 = n1b.reshape(_DEPTH, 1, _D)
    qkv_b3 = qkv_b.reshape(_DEPTH, 1, 3 * _D)
    proj_b3 = proj_b.reshape(_DEPTH, 1, _D)
    n2g3 = n2g.reshape(_DEPTH, 1, _D)
    n2b3 = n2b.reshape(_DEPTH, 1, _D)
    fc1_b3 = fc1_b.reshape(_DEPTH, 1, _MLP)
    fc2_b3 = fc2_b.reshape(_DEPTH, 1, _D)

    def phase(xx, mask, l0, nlayers, masked):
        return _run_phase(xx, mask, l0, nlayers, masked, n1g3, n1b3, qkv_w,
                          qkv_b3, proj_w, proj_b3, n2g3, n2b3, fc1_w, fc1_b3,
                          fc2_w, fc2_b3)

    dense_mask = jnp.ones((_B, 1, _S), jnp.float32)
    x = phase(x, dense_mask, 0, _PRUNE_AFTER, masked=False)

    packed, valid = _pack(x)
    packed = phase(packed, valid, _PRUNE_AFTER, _DEPTH - _PRUNE_AFTER,
                   masked=True)

    cls = packed[:, 0, :]
    return _head(cls, norm_g.reshape(1, _D), norm_b.reshape(1, _D),
                 head_w, head_b.reshape(1, _NCLS))


# speedup vs baseline: 1.1874x; 1.0628x over previous
import math

import functools

import jax
import jax.numpy as jnp
from jax.experimental import pallas as pl
from jax.experimental.pallas import tpu as pltpu

_B = 8
_IMG = 224
_PATCH = 16
_GRID = _IMG // _PATCH
_NPATCH = _GRID * _GRID
_S = _NPATCH + 1
_D = 768
_H = 12
_HD = _D // _H
_DEPTH = 12
_PRUNE_AFTER = 4
_MLP = 4 * _D
_NCLS = 1000
_NKEEP = 98
_KP = 104
_PDIM = 3 * _PATCH * _PATCH
_EPS = 1e-6
_ISQRT2 = 0.7071067811865476
_ASCALE = 1.0 / math.sqrt(_HD)


def _mm(a, b, precision=None):
    return jax.lax.dot_general(a, b, (((1,), (0,)), ((), ())),
                               preferred_element_type=jnp.float32,
                               precision=precision)


def _mmT(a, b, precision=None):
    return jax.lax.dot_general(a, b, (((1,), (1,)), ((), ())),
                               preferred_element_type=jnp.float32,
                               precision=precision)


def _split(a):
    hi = a.astype(jnp.bfloat16)
    lo = (a - hi.astype(jnp.float32)).astype(jnp.bfloat16)
    return hi, lo


def _mm3T(a, w):
    return _mmT(a.astype(jnp.bfloat16), w.astype(jnp.bfloat16))


def _mm3(a, w):
    return _mm(a.astype(jnp.bfloat16), w.astype(jnp.bfloat16))


def _ln(x, g, b):
    m = jnp.mean(x, axis=-1, keepdims=True)
    xc = x - m
    v = jnp.mean(xc * xc, axis=-1, keepdims=True)
    return xc / jnp.sqrt(v + _EPS) * g + b


def _gelu(x):
    return 0.5 * x * (1.0 + jax.lax.erf(x * _ISQRT2))


def _front_kernel(p_ref, pw_ref, pb_ref, cls_ref, pos0_ref, posr_ref, out_ref):
    emb = _mm3T(p_ref[0], pw_ref[...]) + pb_ref[...] + posr_ref[...]
    row0 = cls_ref[...] + pos0_ref[...]
    out_ref[0] = jnp.concatenate([row0, emb], axis=0)


def _front(p, patch_w, patch_b, cls_tok, pos0, posr):
    return pl.pallas_call(
        _front_kernel,
        grid=(_B,),
        in_specs=[
            pl.BlockSpec((1, _NPATCH, _PDIM), lambda b: (b, 0, 0)),
            pl.BlockSpec((_D, _PDIM), lambda b: (0, 0)),
            pl.BlockSpec((1, _D), lambda b: (0, 0)),
            pl.BlockSpec((1, _D), lambda b: (0, 0)),
            pl.BlockSpec((1, _D), lambda b: (0, 0)),
            pl.BlockSpec((_NPATCH, _D), lambda b: (0, 0)),
        ],
        out_specs=pl.BlockSpec((1, _S, _D), lambda b: (b, 0, 0)),
        out_shape=jax.ShapeDtypeStruct((_B, _S, _D), jnp.float32),
    )(p, patch_w, patch_b, cls_tok, pos0, posr)


def _block_kernel(x_ref, m_ref, g1_ref, b1_ref, qb_ref, pb_ref, g2_ref,
                  b2_ref, c1_ref, c2_ref, qw_ref, pw_ref, w1_ref, w2_ref,
                  out_ref, *, masked):
    x = x_ref[0]
    h = _ln(x, g1_ref[0], b1_ref[0])
    qkv = _mm3T(h, qw_ref[0]) + qb_ref[0]
    neg = (1.0 - m_ref[0]) * (-1e30) if masked else None
    outs = []
    for i in range(_H):
        qh = qkv[:, i * _HD:(i + 1) * _HD]
        kh = qkv[:, _D + i * _HD:_D + (i + 1) * _HD]
        vh = qkv[:, 2 * _D + i * _HD:2 * _D + (i + 1) * _HD]
        lg = _mm3T(qh, kh) * _ASCALE
        if masked:
            lg = lg + neg
        lg = lg - jnp.max(lg, axis=-1, keepdims=True)
        e = jnp.exp(lg)
        a = e / jnp.sum(e, axis=-1, keepdims=True)
        outs.append(_mm3(a, vh))
    o = jnp.concatenate(outs, axis=1)
    x = x + _mm3T(o, pw_ref[0]) + pb_ref[0]
    h2 = _ln(x, g2_ref[0], b2_ref[0])
    mh = _gelu(_mm3T(h2, w1_ref[0]) + c1_ref[0])
    out_ref[0] = x + _mm3T(mh, w2_ref[0]) + c2_ref[0]


def _run_phase(x, mask, l0, nlayers, masked, n1g, n1b, qkv_w, qkv_b, proj_w,
               proj_b, n2g, n2b, fc1_w, fc1_b, fc2_w, fc2_b):
    seq = x.shape[1]

    def w3(shape):
        return pl.BlockSpec((1,) + shape, lambda l, b: (l0 + l, 0, 0))

    xspec = pl.BlockSpec((1, seq, _D), lambda l, b: (b, 0, 0))
    return pl.pallas_call(
        functools.partial(_block_kernel, masked=masked),
        grid=(nlayers, _B),
        in_specs=[
            xspec,
            pl.BlockSpec((1, 1, seq), lambda l, b: (b, 0, 0)),
            w3((1, _D)), w3((1, _D)), w3((1, 3 * _D)), w3((1, _D)),
            w3((1, _D)), w3((1, _D)), w3((1, _MLP)), w3((1, _D)),
            w3((3 * _D, _D)), w3((_D, _D)), w3((_MLP, _D)), w3((_D, _MLP)),
        ],
        out_specs=xspec,
        out_shape=jax.ShapeDtypeStruct((_B, seq, _D), jnp.float32),
        input_output_aliases={0: 0},
        compiler_params=pltpu.CompilerParams(
            dimension_semantics=("arbitrary", "arbitrary"),
            vmem_limit_bytes=100 * 1024 * 1024,
        ),
    )(x, mask, n1g, n1b, qkv_b, proj_b, n2g, n2b, fc1_b, fc2_b,
      qkv_w, proj_w, fc1_w, fc2_w)


def _transpose_col(col, n):
    i_col = jax.lax.broadcasted_iota(jnp.int32, (n, 1), 0)
    j_row = jax.lax.broadcasted_iota(jnp.int32, (1, n), 1)
    return jnp.sum(jnp.where(i_col == j_row, col, 0.0), axis=0, keepdims=True)


def _pack_kernel(x_ref, out_ref, valid_ref):
    x = x_ref[0]
    s_col = jnp.sqrt(jnp.sum(x * x, axis=1, keepdims=True))
    s_row = _transpose_col(s_col, _S)
    i_col = jax.lax.broadcasted_iota(jnp.int32, (_S, 1), 0)
    j_row = jax.lax.broadcasted_iota(jnp.int32, (1, _S), 1)
    beats = (s_row > s_col) | ((s_row == s_col) & (j_row < i_col))
    rank = jnp.sum(beats.astype(jnp.float32), axis=1, keepdims=True)
    keep_col = ((rank < float(_NKEEP)) | (i_col == 0)).astype(jnp.float32)
    keep_row = _transpose_col(keep_col, _S)
    count = jnp.sum(keep_col)
    below = (j_row < i_col).astype(jnp.float32)
    pos_col = jnp.sum(keep_row * below, axis=1, keepdims=True)
    pos_row = _transpose_col(pos_col, _S)
    p_col = jax.lax.broadcasted_iota(jnp.int32, (_KP, 1), 0).astype(jnp.float32)
    sel = ((p_col == pos_row) & (keep_row > 0.5)).astype(jnp.float32)
    x_hi, x_lo = _split(x)
    out_ref[0] = _mm(sel, x_hi) + _mm(sel, x_lo)
    kp_row = jax.lax.broadcasted_iota(jnp.int32, (1, _KP), 1).astype(jnp.float32)
    valid_ref[0] = (kp_row < count).astype(jnp.float32)


def _pack(x):
    return pl.pallas_call(
        _pack_kernel,
        grid=(_B,),
        in_specs=[pl.BlockSpec((1, _S, _D), lambda b: (b, 0, 0))],
        out_specs=[
            pl.BlockSpec((1, _KP, _D), lambda b: (b, 0, 0)),
            pl.BlockSpec((1, 1, _KP), lambda b: (b, 0, 0)),
        ],
        out_shape=[
            jax.ShapeDtypeStruct((_B, _KP, _D), jnp.float32),
            jax.ShapeDtypeStruct((_B, 1, _KP), jnp.float32),
        ],
    )(x)


def _head_kernel(x_ref, g_ref, b_ref, w_ref, hb_ref, out_ref):
    h = _ln(x_ref[...], g_ref[...], b_ref[...])
    out_ref[...] = _mm3T(h, w_ref[...]) + hb_ref[...]


def _head(cls, norm_g, norm_b, head_w, head_b):
    return pl.pallas_call(
        _head_kernel,
        out_shape=jax.ShapeDtypeStruct((_B, _NCLS), jnp.float32),
    )(cls, norm_g, norm_b, head_w, head_b)


def kernel(images, patch_w, patch_b, cls_token, pos_embed, n1g, n1b, qkv_w,
           qkv_b, proj_w, proj_b, n2g, n2b, fc1_w, fc1_b, fc2_w, fc2_b,
           norm_g, norm_b, head_w, head_b):
    p = images.reshape(_B, 3, _GRID, _PATCH, _GRID, _PATCH)
    p = p.transpose(0, 2, 4, 1, 3, 5).reshape(_B, _NPATCH, _PDIM)
    pos = pos_embed.reshape(_S, _D)
    x = _front(p, patch_w, patch_b.reshape(1, _D), cls_token.reshape(1, _D),
               pos[0:1], pos[1:])

    n1g3 = n1g.reshape(_DEPTH, 1, _D)
    n1b3 = n1b.reshape(_DEPTH, 1, _D)
    qkv_b3 = qkv_b.reshape(_DEPTH, 1, 3 * _D)
    proj_b3 = proj_b.reshape(_DEPTH, 1, _D)
    n2g3 = n2g.reshape(_DEPTH, 1, _D)
    n2b3 = n2b.reshape(_DEPTH, 1, _D)
    fc1_b3 = fc1_b.reshape(_DEPTH, 1, _MLP)
    fc2_b3 = fc2_b.reshape(_DEPTH, 1, _D)

    def phase(xx, mask, l0, nlayers, masked):
        return _run_phase(xx, mask, l0, nlayers, masked, n1g3, n1b3, qkv_w,
                          qkv_b3, proj_w, proj_b3, n2g3, n2b3, fc1_w, fc1_b3,
                          fc2_w, fc2_b3)

    dense_mask = jnp.ones((_B, 1, _S), jnp.float32)
    x = phase(x, dense_mask, 0, _PRUNE_AFTER, masked=False)

    packed, valid = _pack(x)
    packed = phase(packed, valid, _PRUNE_AFTER, _DEPTH - _PRUNE_AFTER,
                   masked=True)

    cls = packed[:, 0, :]
    return _head(cls, norm_g.reshape(1, _D), norm_b.reshape(1, _D),
                 head_w, head_b.reshape(1, _NCLS))
